# Initial kernel scaffold; baseline (speedup 1.0000x reference)
import jax
import jax.numpy as jnp
from jax.experimental import pallas as pl


def kernel(class_indices, object_positions, description_encodings, params):
    raise NotImplementedError("write your pallas kernel here")


# speedup vs baseline: 4.2547x; 4.2547x over previous
import functools

import jax
import jax.numpy as jnp
from jax.experimental import pallas as pl
from jax.experimental.pallas import tpu as pltpu

_E = 128
_V = 1001
_B = 4
_N = 512
_K = 16

_F32 = jnp.float32
_HIGH = jax.lax.Precision.HIGHEST


def _dot(x, w):
    return jax.lax.dot_general(
        x, w, (((x.ndim - 1,), (0,)), ((), ())),
        preferred_element_type=_F32, precision=_HIGH)


def _small_linear(x, w, b):
    s = None
    for d in range(w.shape[0]):
        t = x[:, d:d + 1] * w[d:d + 1, :]
        s = t if s is None else s + t
    return s + b


def _mlp_chain(x, layers):
    n = len(layers)
    for i, (w, b) in enumerate(layers):
        x = _dot(x, w) + b
        if i < n - 1:
            x = jnp.maximum(x, 0.0)
    return x


def _body(cls_ref, pos_ref, desc_ref, table_ref,
          pw0, pb0, pw1, pb1, pw2, pb2, pw3, pb3,
          ew0, eb0, ew1, eb1,
          fw, fb,
          rw0, rb0, rw1, rb1, rw2, rb2, rw3, rb3,
          tw0, tb0, tw1, tb1,
          cw0, cb0, cw1, cb1, cw2, cb2,
          ow0, ob0, ow1, ob1, ow2, ob2,
          feats_out, ref_out, tcls_out, ocls_out, ooff_out):
    cls = cls_ref[0]
    onehot_v = (jax.lax.broadcasted_iota(jnp.int32, (_N, _V), 1)
                == cls).astype(_F32)
    class_emb = _dot(onehot_v, table_ref[...])

    p = pos_ref[0]
    p = jnp.maximum(_small_linear(p, pw0[...], pb0[...]), 0.0)
    p = jnp.maximum(_small_linear(p, pw1[...], pb1[...]), 0.0)
    p = jnp.maximum(_small_linear(p, pw2[...], pb2[...]), 0.0)
    pos_emb = _small_linear(p, pw3[...], pb3[...])

    desc = desc_ref[0]
    desc_b = jnp.broadcast_to(desc, (_N, _E))
    x = jnp.concatenate([class_emb + pos_emb, desc_b], axis=1)

    iota_j = jax.lax.broadcasted_iota(jnp.int32, (_N, _N), 1)
    iota_i = jax.lax.broadcasted_iota(jnp.int32, (_N, _N), 0)
    eye = iota_i == iota_j
    inf = jnp.float32(jnp.inf)

    outs = []
    for ew, eb in ((ew0, eb0), (ew1, eb1)):
        w = ew[...]
        w1 = w[:2 * _E, :]
        w2 = w[2 * _E:, :]
        a = _dot(x, w1 - w2) + eb[...]
        bb = _dot(x, w2)

        gram = jax.lax.dot_general(
            x, x, (((1,), (1,)), ((), ())),
            preferred_element_type=_F32, precision=_HIGH)
        sq_col = jnp.sum(x * x, axis=1, keepdims=True)
        sq_row = jnp.sum(jnp.where(eye, gram, 0.0), axis=0,
                         keepdims=True)
        dist = sq_col + sq_row - 2.0 * gram

        acc = None
        for _ in range(_K):
            m = jnp.min(dist, axis=1, keepdims=True)
            cand = jnp.where(dist <= m, iota_j, _N)
            jmin = jnp.min(cand, axis=1, keepdims=True)
            onehot = iota_j == jmin
            g = _dot(jnp.where(onehot, 1.0, 0.0), bb)
            acc = g if acc is None else jnp.maximum(acc, g)
            dist = jnp.where(onehot, inf, dist)
        x = a + acc
        outs.append(x)

    cat = jnp.concatenate([outs[0], outs[1], desc_b], axis=1)
    feats = _dot(cat, fw[...]) + fb[...]
    feats_out[0] = feats

    ref_out[0] = _mlp_chain(
        feats, [(rw0[...], rb0[...]), (rw1[...], rb1[...]),
                (rw2[...], rb2[...]), (rw3[...], rb3[...])])

    tcls_out[0] = _mlp_chain(
        desc, [(tw0[...], tb0[...]), (tw1[...], tb1[...])])

    ocls_out[0] = _mlp_chain(
        feats, [(cw0[...], cb0[...]), (cw1[...], cb1[...]),
                (cw2[...], cb2[...])])

    ooff_out[0] = _mlp_chain(
        feats, [(ow0[...], ob0[...]), (ow1[...], ob1[...]),
                (ow2[...], ob2[...])])


@jax.jit
def kernel(class_indices, object_positions, description_encodings, params):
    cls3 = class_indices.astype(jnp.int32).reshape(_B, _N, 1)
    desc3 = description_encodings.reshape(_B, 1, _E)

    weights = []
    for w, b in (params["pos_mlp"]
                 + params["edge_mlps"][0] + params["edge_mlps"][1]
                 + params["mlp_features"]
                 + params["mlp_object_ref"]
                 + params["mlp_target_class"]
                 + params["mlp_object_class"]
                 + params["mlp_object_offset"]):
        weights.append(w)
        weights.append(b.reshape(1, -1))

    def batch_spec(shape):
        nd = len(shape)
        return pl.BlockSpec((1,) + shape[1:],
                            lambda b, _nd=nd: (b,) + (0,) * (_nd - 1))

    def full_spec(shape):
        nd = len(shape)
        return pl.BlockSpec(shape, lambda b, _nd=nd: (0,) * _nd)

    in_specs = [
        batch_spec((_B, _N, 1)),
        batch_spec((_B, _N, 2)),
        batch_spec((_B, 1, _E)),
        full_spec((_V, _E)),
    ] + [full_spec(w.shape) for w in weights]

    out_specs = [
        batch_spec((_B, _N, 2 * _E)),
        batch_spec((_B, _N, 1)),
        batch_spec((_B, 1, _V)),
        batch_spec((_B, _N, _V)),
        batch_spec((_B, _N, 2)),
    ]
    out_shape = [
        jax.ShapeDtypeStruct((_B, _N, 2 * _E), _F32),
        jax.ShapeDtypeStruct((_B, _N, 1), _F32),
        jax.ShapeDtypeStruct((_B, 1, _V), _F32),
        jax.ShapeDtypeStruct((_B, _N, _V), _F32),
        jax.ShapeDtypeStruct((_B, _N, 2), _F32),
    ]

    feats, oref, tcls, ocls, ooff = pl.pallas_call(
        _body,
        grid=(_B,),
        in_specs=in_specs,
        out_specs=out_specs,
        out_shape=out_shape,
        compiler_params=pltpu.CompilerParams(
            dimension_semantics=("arbitrary",)),
    )(cls3, object_positions, desc3, params["class_table"], *weights)

    return (feats, oref[..., 0], tcls[:, 0, :], ocls, ooff)
